# baseline (speedup 1.0000x reference)
import functools

import jax
import jax.numpy as jnp
from jax import lax
from jax.experimental import pallas as pl
from jax.experimental.pallas import tpu as pltpu
from jax.experimental.pallas import tpu_sc as plsc

_T = 128


def _make_sc_kernel(B, F, L):
    info = plsc.get_sparse_core_info()
    NC, NS, NL = info.num_cores, info.num_subcores, info.num_lanes
    NW = NC * NS
    NF = F // _T
    NLC = L // _T
    NBH = NW // (NF * NLC)
    BPW = B // NBH

    mesh = plsc.VectorSubcoreMesh(core_axis_name="c", subcore_axis_name="s")

    @functools.partial(
        pl.kernel,
        out_type=jax.ShapeDtypeStruct((B, F, L), jnp.float32),
        mesh=mesh,
        scratch_types=[
            pltpu.VMEM((_T, _T), jnp.float32),
            pltpu.VMEM((_T, _T), jnp.float32),
            pltpu.SemaphoreType.DMA,
        ],
        compiler_params=pltpu.CompilerParams(needs_layout_passes=False),
    )
    def sc_fn(table_hbm, out_hbm, in_v, t_v, sem):
        wid = lax.axis_index("s") * NC + lax.axis_index("c")
        fi = wid % NF
        li = (wid // NF) % NLC
        bh = wid // (NF * NLC)
        l0 = li * _T
        f0 = fi * _T

        pltpu.sync_copy(
            table_hbm.at[pl.ds(l0, _T), pl.ds(f0, _T)], in_v
        )

        rows = [k * NL + lax.iota(jnp.int32, NL) for k in range(_T // NL)]

        @plsc.parallel_loop(0, _T, step=1)
        def _(f):
            cols = jnp.full((NL,), 0, jnp.int32) + f
            for k in range(_T // NL):
                t_v[f, pl.ds(k * NL, NL)] = plsc.load_gather(in_v, [rows[k], cols])

        copies = [
            pltpu.make_async_copy(
                t_v,
                out_hbm.at[bh * BPW + db, pl.ds(f0, _T), pl.ds(l0, _T)],
                sem,
            )
            for db in range(BPW)
        ]
        for cp in copies:
            cp.start()
        for cp in copies:
            cp.wait()

    return sc_fn


def kernel(x, mask, row_embed):
    B = x.shape[0]
    F = x.shape[1]
    L = x.shape[-1]
    return _make_sc_kernel(B, F, L)(row_embed)


# speedup vs baseline: 6.4643x; 6.4643x over previous
import jax
import jax.numpy as jnp
from jax.experimental import pallas as pl


def _pos_embed_kernel(emb_ref, out_ref):
    m = pl.program_id(1)
    LS = emb_ref.shape[0]
    t = emb_ref[...].T
    out_ref[:, :, pl.ds(m * LS, LS)] = jnp.broadcast_to(
        t[None], (out_ref.shape[0],) + t.shape
    )


def kernel(x, mask, row_embed):
    B = x.shape[0]
    F = x.shape[1]
    L = x.shape[-1]
    LT = 512
    LS = 256
    return pl.pallas_call(
        _pos_embed_kernel,
        grid=(L // LT, LT // LS),
        in_specs=[pl.BlockSpec((LS, F), lambda l, m: (l * (LT // LS) + m, 0))],
        out_specs=pl.BlockSpec((B, F, LT), lambda l, m: (0, 0, l)),
        out_shape=jax.ShapeDtypeStruct((B, F, L), jnp.float32),
    )(row_embed)


# speedup vs baseline: 8.3625x; 1.2936x over previous
import jax
import jax.numpy as jnp
from jax.experimental import pallas as pl
from jax.experimental.pallas import tpu as pltpu


def _pos_embed_kernel(emb_ref, out_ref, t_ref, sems):
    i = pl.program_id(0)
    n = pl.num_programs(0)
    B, F, L = out_ref.shape
    LT = L // n

    def copies(step):
        return [
            pltpu.make_async_copy(
                t_ref.at[step],
                out_ref.at[b, :, pl.ds(step * LT, LT)],
                sems.at[step, b],
            )
            for b in range(B)
        ]

    t_ref[i] = emb_ref[...].T
    for cp in copies(i):
        cp.start()

    @pl.when(i == n - 1)
    def _():
        for s in range(2):
            for cp in copies(s):
                cp.wait()


def kernel(x, mask, row_embed):
    B = x.shape[0]
    F = x.shape[1]
    L = x.shape[-1]
    LT = 512
    return pl.pallas_call(
        _pos_embed_kernel,
        grid=(L // LT,),
        in_specs=[pl.BlockSpec((LT, F), lambda i: (i, 0))],
        out_specs=pl.BlockSpec(memory_space=pl.ANY),
        out_shape=jax.ShapeDtypeStruct((B, F, L), jnp.float32),
        scratch_shapes=[
            pltpu.VMEM((L // LT, F, LT), jnp.float32),
            pltpu.SemaphoreType.DMA((L // LT, B)),
        ],
    )(row_embed)
